# Initial kernel scaffold; baseline (speedup 1.0000x reference)
import functools

import jax
import jax.numpy as jnp
from jax import lax
from jax.experimental import pallas as pl
from jax.experimental.pallas import tpu as pltpu
from jax.experimental.pallas import tpu_sc as plsc

VOCAB_ROWS = 30000
DIM = 100

NUM_CORES = 2
NUM_SUBCORES = 16
LANES = 16
NUM_WORKERS = NUM_CORES * NUM_SUBCORES

ROW_BLOCK = 3000


def _tw_body(table_ref, w_ref, b_ref, out_ref):
    acc = jnp.sum(table_ref[...] * w_ref[...], axis=1, keepdims=True)
    out_ref[...] = acc + b_ref[0]


def _precompute_tw(table, W, b):
    return pl.pallas_call(
        _tw_body,
        grid=(VOCAB_ROWS // ROW_BLOCK,),
        in_specs=[
            pl.BlockSpec((ROW_BLOCK, DIM), lambda i: (i, 0)),
            pl.BlockSpec((1, DIM), lambda i: (0, 0)),
            pl.BlockSpec(memory_space=pltpu.SMEM),
        ],
        out_specs=pl.BlockSpec((ROW_BLOCK, 1), lambda i: (i, 0)),
        out_shape=jax.ShapeDtypeStruct((VOCAB_ROWS, 1), jnp.float32),
    )(table, W, b)


def _gather_kernel(n_total):
    n_per_w = n_total // NUM_WORKERS
    mesh = plsc.VectorSubcoreMesh(
        core_axis_name="c", subcore_axis_name="s")

    @functools.partial(
        pl.kernel,
        mesh=mesh,
        out_type=jax.ShapeDtypeStruct((n_total,), jnp.float32),
        scratch_types=[
            pltpu.VMEM((VOCAB_ROWS,), jnp.float32),
            pltpu.VMEM((n_per_w,), jnp.int32),
            pltpu.VMEM((n_per_w,), jnp.float32),
        ],
    )
    def gather(tw_hbm, idx_hbm, out_hbm, tw_v, idx_v, out_v):
        wid = lax.axis_index("s") * NUM_CORES + lax.axis_index("c")
        base = wid * n_per_w
        pltpu.sync_copy(tw_hbm, tw_v)
        pltpu.sync_copy(idx_hbm.at[pl.ds(base, n_per_w)], idx_v)

        def body(i, carry):
            off = i * LANES
            idx16 = idx_v[pl.ds(off, LANES)]
            out_v[pl.ds(off, LANES)] = plsc.load_gather(tw_v, [idx16])
            return carry

        lax.fori_loop(0, n_per_w // LANES, body, 0)
        pltpu.sync_copy(out_v, out_hbm.at[pl.ds(base, n_per_w)])

    return gather


def kernel(x, table, W, b):
    B, L = x.shape
    n_total = B * L
    tw = _precompute_tw(table, W, b)
    flat = _gather_kernel(n_total)(tw.reshape(VOCAB_ROWS), x.reshape(n_total))
    return flat.reshape(B, L, 1)


# speedup vs baseline: 12.3080x; 12.3080x over previous
import functools

import jax
import jax.numpy as jnp
from jax import lax
from jax.experimental import pallas as pl
from jax.experimental.pallas import tpu as pltpu
from jax.experimental.pallas import tpu_sc as plsc

VOCAB_ROWS = 30000
DIM = 100

NUM_CORES = 2
NUM_SUBCORES = 16
LANES = 16
NUM_WORKERS = NUM_CORES * NUM_SUBCORES

ROW_BLOCK = 3000


def _tw_body(table_ref, w_ref, b_ref, out_ref):
    acc = jnp.sum(table_ref[...] * w_ref[...], axis=1, keepdims=True)
    out_ref[...] = acc + b_ref[0]


def _precompute_tw(table, W, b):
    return pl.pallas_call(
        _tw_body,
        grid=(VOCAB_ROWS // ROW_BLOCK,),
        in_specs=[
            pl.BlockSpec((ROW_BLOCK, DIM), lambda i: (i, 0)),
            pl.BlockSpec((1, DIM), lambda i: (0, 0)),
            pl.BlockSpec(memory_space=pltpu.SMEM),
        ],
        out_specs=pl.BlockSpec((ROW_BLOCK, 1), lambda i: (i, 0)),
        out_shape=jax.ShapeDtypeStruct((VOCAB_ROWS, 1), jnp.float32),
    )(table, W, b)


def _gather_kernel(n_total):
    n_per_w = n_total // NUM_WORKERS
    mesh = plsc.VectorSubcoreMesh(
        core_axis_name="c", subcore_axis_name="s",
        num_cores=NUM_CORES, num_subcores=NUM_SUBCORES)

    @functools.partial(
        pl.kernel,
        mesh=mesh,
        out_type=jax.ShapeDtypeStruct((n_total,), jnp.float32),
        scratch_types=[
            pltpu.VMEM((VOCAB_ROWS,), jnp.float32),
            pltpu.VMEM((n_per_w,), jnp.int32),
            pltpu.VMEM((n_per_w,), jnp.float32),
        ],
        compiler_params=pltpu.CompilerParams(
            needs_layout_passes=False, use_tc_tiling_on_sc=False),
    )
    def gather(tw_hbm, idx_hbm, out_hbm, tw_v, idx_v, out_v):
        wid = lax.axis_index("s") * NUM_CORES + lax.axis_index("c")
        base = wid * n_per_w
        pltpu.sync_copy(tw_hbm, tw_v)
        pltpu.sync_copy(idx_hbm.at[pl.ds(base, n_per_w)], idx_v)

        def body(i, carry):
            off = i * LANES
            idx16 = idx_v[pl.ds(off, LANES)]
            out_v[pl.ds(off, LANES)] = plsc.load_gather(tw_v, [idx16])
            return carry

        lax.fori_loop(0, n_per_w // LANES, body, 0)
        pltpu.sync_copy(out_v, out_hbm.at[pl.ds(base, n_per_w)])

    return gather


def kernel(x, table, W, b):
    B, L = x.shape
    n_total = B * L
    tw = _precompute_tw(table, W, b)
    flat = _gather_kernel(n_total)(tw.reshape(VOCAB_ROWS), x.reshape(n_total))
    return flat.reshape(B, L, 1)
